# baseline (speedup 1.0000x reference)
import functools

import jax
import jax.numpy as jnp
from jax import lax
from jax.experimental import pallas as pl
from jax.experimental.pallas import tpu as pltpu
from jax.experimental.pallas import tpu_sc as plsc

_K = 8192
_D = 256
_N = 4096
_BT = 256
_BK = 512
_TG = _N // _BT
_KG = _K // _BK

_NC = 2
_NS = 16
_NW = _NC * _NS
_BPW = _N // _NW
_BINS_PW = _K // _NW
_IDX_TOTAL = 4 * _N


def _argmin_tail(k, d, z2_scr, bv_scr, bi_scr, idx_ref):
    lm = jnp.min(d, axis=1, keepdims=True)
    ii = lax.broadcasted_iota(jnp.int32, d.shape, 1)
    li = jnp.min(jnp.where(d == lm, ii, _BK), axis=1, keepdims=True)
    cand = li + k * _BK

    @pl.when(k == 0)
    def _():
        bv_scr[...] = lm
        bi_scr[...] = cand

    @pl.when(k > 0)
    def _():
        upd = lm < bv_scr[...]
        bi_scr[...] = jnp.where(upd, cand, bi_scr[...])
        bv_scr[...] = jnp.where(upd, lm, bv_scr[...])

    @pl.when(k == _KG - 1)
    def _():
        idx_ref[...] = bi_scr[...]


def _dist_block(r, wt):
    mm = lax.dot_general(r, wt, (((1,), (0,)), ((), ())),
                         preferred_element_type=jnp.float32)
    w2 = jnp.sum(wt * wt, axis=0, keepdims=True)
    return mm, w2


def _argmin_first_body(r_ref, wt_ref, idx_ref, z2_scr, bv_scr, bi_scr):
    k = pl.program_id(1)

    @pl.when(k == 0)
    def _():
        r = r_ref[...]
        z2_scr[...] = jnp.sum(r * r, axis=1, keepdims=True)

    mm, w2 = _dist_block(r_ref[...], wt_ref[...])
    d = (z2_scr[...] + w2) - 2.0 * mm
    _argmin_tail(k, d, z2_scr, bv_scr, bi_scr, idx_ref)


def _argmin_step_body(rp_ref, q_ref, wt_ref, idx_ref, r_out_ref,
                      z2_scr, bv_scr, bi_scr):
    k = pl.program_id(1)

    @pl.when(k == 0)
    def _():
        r = rp_ref[...] - q_ref[...]
        r_out_ref[...] = r
        z2_scr[...] = jnp.sum(r * r, axis=1, keepdims=True)

    mm, w2 = _dist_block(r_out_ref[...], wt_ref[...])
    d = (z2_scr[...] + w2) - 2.0 * mm
    _argmin_tail(k, d, z2_scr, bv_scr, bi_scr, idx_ref)


_tok_spec = pl.BlockSpec((_BT, _D), lambda t, k: (t, 0))
_wt_spec = pl.BlockSpec((_D, _BK), lambda t, k: (0, k))
_idx_spec = pl.BlockSpec((_BT, 1), lambda t, k: (t, 0))
_scratch = [
    pltpu.VMEM((_BT, 1), jnp.float32),
    pltpu.VMEM((_BT, 1), jnp.float32),
    pltpu.VMEM((_BT, 1), jnp.int32),
]

_argmin_first = pl.pallas_call(
    _argmin_first_body,
    grid=(_TG, _KG),
    in_specs=[_tok_spec, _wt_spec],
    out_specs=_idx_spec,
    out_shape=jax.ShapeDtypeStruct((_N, 1), jnp.int32),
    scratch_shapes=_scratch,
)

_argmin_step = pl.pallas_call(
    _argmin_step_body,
    grid=(_TG, _KG),
    in_specs=[_tok_spec, _tok_spec, _wt_spec],
    out_specs=[_idx_spec, _tok_spec],
    out_shape=[
        jax.ShapeDtypeStruct((_N, 1), jnp.int32),
        jax.ShapeDtypeStruct((_N, _D), jnp.float32),
    ],
    scratch_shapes=_scratch,
)


def _gather_body(table_ref, idx_ref, out_ref, idx_v, rows_v, sem):
    wid = lax.axis_index("s") * _NC + lax.axis_index("c")
    base = wid * _BPW
    pltpu.sync_copy(idx_ref.at[pl.ds(base, _BPW)], idx_v)
    pltpu.async_copy(table_ref.at[idx_v], rows_v, sem).wait()
    pltpu.sync_copy(rows_v, out_ref.at[pl.ds(base, _BPW)])


@functools.cache
def _sc_mesh():
    return plsc.VectorSubcoreMesh(core_axis_name="c", subcore_axis_name="s",
                                  num_cores=_NC, num_subcores=_NS)


@functools.cache
def _gather_kernel():
    return pl.kernel(
        _gather_body,
        out_type=jax.ShapeDtypeStruct((_N, _D), jnp.float32),
        mesh=_sc_mesh(),
        scratch_types=[
            pltpu.VMEM((_BPW,), jnp.int32),
            pltpu.VMEM((_BPW, _D), jnp.float32),
            pltpu.SemaphoreType.DMA,
        ],
    )


def _bincount_body(idx_ref, out_ref, idx_v, hist_v, acc_v):
    wid = lax.axis_index("s") * _NC + lax.axis_index("c")
    base_bin = wid * _BINS_PW
    lanes = lax.iota(jnp.int32, 16)
    zeros = jnp.zeros((16,), jnp.float32)
    ones = jnp.ones((16,), jnp.float32)

    def zero_body(i, c):
        hist_v[pl.ds(i * 16, 16)] = zeros
        return c

    lax.fori_loop(0, 16 * _BINS_PW // 16, zero_body, 0)

    pltpu.sync_copy(idx_ref, idx_v)

    def scat_body(i, c):
        v = idx_v[pl.ds(i * 16, 16)]
        rel = v - base_bin
        m = (rel >= 0) & (rel < _BINS_PW)
        relc = jnp.where(m, rel, 0)
        flat = lanes * _BINS_PW + relc
        plsc.addupdate_scatter(hist_v, [flat], ones, mask=m)
        return c

    lax.fori_loop(0, _IDX_TOTAL // 16, scat_body, 0)

    for c in range(_BINS_PW // 16):
        acc = hist_v[pl.ds(c * 16, 16)]
        for r in range(1, 16):
            acc = acc + hist_v[pl.ds(r * _BINS_PW + c * 16, 16)]
        acc_v[pl.ds(c * 16, 16)] = acc

    pltpu.sync_copy(acc_v, out_ref.at[pl.ds(base_bin, _BINS_PW)])


@functools.cache
def _bincount_kernel():
    return pl.kernel(
        _bincount_body,
        out_type=jax.ShapeDtypeStruct((_K,), jnp.float32),
        mesh=_sc_mesh(),
        scratch_types=[
            pltpu.VMEM((_IDX_TOTAL,), jnp.int32),
            pltpu.VMEM((16 * _BINS_PW,), jnp.float32),
            pltpu.VMEM((_BINS_PW,), jnp.float32),
        ],
        compiler_params=pltpu.CompilerParams(needs_layout_passes=False),
    )


def _perp_body(c_ref, o_ref):
    p = c_ref[...] / float(_IDX_TOTAL)
    ent = p * jnp.log(jnp.maximum(p, 1e-10))
    s = jnp.sum(ent, axis=(0, 1), keepdims=True)
    o_ref[...] = jnp.exp(-s)


_perp = pl.pallas_call(
    _perp_body,
    out_shape=jax.ShapeDtypeStruct((1, 1), jnp.float32),
)


def kernel(z, codebook):
    B, C, H, W = z.shape
    flat_z = jnp.transpose(z, (0, 2, 3, 1)).reshape(-1, C)
    wt = codebook.T

    gather = _gather_kernel()
    idx0 = _argmin_first(flat_z, wt)
    q0 = gather(codebook, idx0.reshape(-1))
    idx1, r1 = _argmin_step(flat_z, q0, wt)
    q1 = gather(codebook, idx1.reshape(-1))
    idx2, r2 = _argmin_step(r1, q1, wt)
    q2 = gather(codebook, idx2.reshape(-1))
    idx3, r3 = _argmin_step(r2, q2, wt)
    q3 = gather(codebook, idx3.reshape(-1))

    r4 = r3 - q3
    cum1 = q1 + q0
    cum2 = q2 + cum1
    cum3 = q3 + cum2
    st = cum3 + (r4 - cum3)

    idx_all = jnp.concatenate(
        [idx0.reshape(-1), idx1.reshape(-1), idx2.reshape(-1), idx3.reshape(-1)],
        axis=0)
    counts = _bincount_kernel()(idx_all)
    perp = _perp(counts.reshape(64, 128)).reshape(())

    def unflat(x):
        return x.reshape(B, H, W, C).transpose(0, 3, 1, 2)

    return (unflat(q0), unflat(cum1), unflat(cum2), unflat(cum3), unflat(st),
            idx0.reshape(-1), idx1.reshape(-1), idx2.reshape(-1),
            idx3.reshape(-1), perp)


# speedup vs baseline: 1.2848x; 1.2848x over previous
import functools

import jax
import jax.numpy as jnp
from jax import lax
from jax.experimental import pallas as pl
from jax.experimental.pallas import tpu as pltpu
from jax.experimental.pallas import tpu_sc as plsc

_K = 8192
_D = 256
_N = 4096
_BT = 256
_BK = 512
_TG = _N // _BT
_KG = _K // _BK

_NC = 2
_NS = 16
_NW = _NC * _NS
_BPW = _N // _NW
_BINS_PW = _K // _NW
_IDX_TOTAL = 4 * _N


def _argmin_tail(k, d, z2_scr, bv_scr, bi_scr, idx_ref):
    lm = jnp.min(d, axis=1, keepdims=True)
    ii = lax.broadcasted_iota(jnp.int32, d.shape, 1)
    li = jnp.min(jnp.where(d == lm, ii, _BK), axis=1, keepdims=True)
    cand = li + k * _BK

    @pl.when(k == 0)
    def _():
        bv_scr[...] = lm
        bi_scr[...] = cand

    @pl.when(k > 0)
    def _():
        upd = lm < bv_scr[...]
        bi_scr[...] = jnp.where(upd, cand, bi_scr[...])
        bv_scr[...] = jnp.where(upd, lm, bv_scr[...])

    @pl.when(k == _KG - 1)
    def _():
        idx_ref[...] = bi_scr[...]


def _dist_block(r, wt):
    mm = lax.dot_general(r, wt, (((1,), (0,)), ((), ())),
                         preferred_element_type=jnp.float32)
    w2 = jnp.sum(wt * wt, axis=0, keepdims=True)
    return mm, w2


def _argmin_first_body(r_ref, wt_ref, idx_ref, z2_scr, bv_scr, bi_scr):
    k = pl.program_id(1)

    @pl.when(k == 0)
    def _():
        r = r_ref[...]
        z2_scr[...] = jnp.sum(r * r, axis=1, keepdims=True)

    mm, w2 = _dist_block(r_ref[...], wt_ref[:, pl.ds(k * _BK, _BK)])
    d = (z2_scr[...] + w2) - 2.0 * mm
    _argmin_tail(k, d, z2_scr, bv_scr, bi_scr, idx_ref)


def _argmin_step_body(rp_ref, q_ref, wt_ref, idx_ref, r_out_ref,
                      z2_scr, bv_scr, bi_scr):
    k = pl.program_id(1)

    @pl.when(k == 0)
    def _():
        r = rp_ref[...] - q_ref[...]
        r_out_ref[...] = r
        z2_scr[...] = jnp.sum(r * r, axis=1, keepdims=True)

    mm, w2 = _dist_block(r_out_ref[...], wt_ref[:, pl.ds(k * _BK, _BK)])
    d = (z2_scr[...] + w2) - 2.0 * mm
    _argmin_tail(k, d, z2_scr, bv_scr, bi_scr, idx_ref)


_tok_spec = pl.BlockSpec((_BT, _D), lambda t, k: (t, 0))
_wt_spec = pl.BlockSpec((_D, _K), lambda t, k: (0, 0))
_idx_spec = pl.BlockSpec((_BT, 1), lambda t, k: (t, 0))
_scratch = [
    pltpu.VMEM((_BT, 1), jnp.float32),
    pltpu.VMEM((_BT, 1), jnp.float32),
    pltpu.VMEM((_BT, 1), jnp.int32),
]

_argmin_first = pl.pallas_call(
    _argmin_first_body,
    grid=(_TG, _KG),
    in_specs=[_tok_spec, _wt_spec],
    out_specs=_idx_spec,
    out_shape=jax.ShapeDtypeStruct((_N, 1), jnp.int32),
    scratch_shapes=_scratch,
)

_argmin_step = pl.pallas_call(
    _argmin_step_body,
    grid=(_TG, _KG),
    in_specs=[_tok_spec, _tok_spec, _wt_spec],
    out_specs=[_idx_spec, _tok_spec],
    out_shape=[
        jax.ShapeDtypeStruct((_N, 1), jnp.int32),
        jax.ShapeDtypeStruct((_N, _D), jnp.float32),
    ],
    scratch_shapes=_scratch,
)


def _gather_body(table_ref, idx_ref, out_ref, idx_v, rows_v, sem):
    wid = lax.axis_index("s") * _NC + lax.axis_index("c")
    base = wid * _BPW
    pltpu.sync_copy(idx_ref.at[pl.ds(base, _BPW)], idx_v)
    pltpu.async_copy(table_ref.at[idx_v], rows_v, sem).wait()
    pltpu.sync_copy(rows_v, out_ref.at[pl.ds(base, _BPW)])


@functools.cache
def _sc_mesh():
    return plsc.VectorSubcoreMesh(core_axis_name="c", subcore_axis_name="s",
                                  num_cores=_NC, num_subcores=_NS)


@functools.cache
def _gather_kernel():
    return pl.kernel(
        _gather_body,
        out_type=jax.ShapeDtypeStruct((_N, _D), jnp.float32),
        mesh=_sc_mesh(),
        scratch_types=[
            pltpu.VMEM((_BPW,), jnp.int32),
            pltpu.VMEM((_BPW, _D), jnp.float32),
            pltpu.SemaphoreType.DMA,
        ],
    )


def _bincount_body(idx_ref, out_ref, idx_v, hist_v, acc_v):
    wid = lax.axis_index("s") * _NC + lax.axis_index("c")
    base_bin = wid * _BINS_PW
    lanes = lax.iota(jnp.int32, 16)
    zeros = jnp.zeros((16,), jnp.float32)
    ones = jnp.ones((16,), jnp.float32)

    def zero_body(i, c):
        hist_v[pl.ds(i * 16, 16)] = zeros
        return c

    lax.fori_loop(0, 16 * _BINS_PW // 16, zero_body, 0)

    pltpu.sync_copy(idx_ref, idx_v)

    def scat_body(i, c):
        v = idx_v[pl.ds(i * 16, 16)]
        rel = v - base_bin
        m = (rel >= 0) & (rel < _BINS_PW)
        relc = jnp.where(m, rel, 0)
        flat = lanes * _BINS_PW + relc
        plsc.addupdate_scatter(hist_v, [flat], ones, mask=m)
        return c

    lax.fori_loop(0, _IDX_TOTAL // 16, scat_body, 0)

    for c in range(_BINS_PW // 16):
        acc = hist_v[pl.ds(c * 16, 16)]
        for r in range(1, 16):
            acc = acc + hist_v[pl.ds(r * _BINS_PW + c * 16, 16)]
        acc_v[pl.ds(c * 16, 16)] = acc

    pltpu.sync_copy(acc_v, out_ref.at[pl.ds(base_bin, _BINS_PW)])


@functools.cache
def _bincount_kernel():
    return pl.kernel(
        _bincount_body,
        out_type=jax.ShapeDtypeStruct((_K,), jnp.float32),
        mesh=_sc_mesh(),
        scratch_types=[
            pltpu.VMEM((_IDX_TOTAL,), jnp.int32),
            pltpu.VMEM((16 * _BINS_PW,), jnp.float32),
            pltpu.VMEM((_BINS_PW,), jnp.float32),
        ],
        compiler_params=pltpu.CompilerParams(needs_layout_passes=False),
    )


def _perp_body(c_ref, o_ref):
    p = c_ref[...] / float(_IDX_TOTAL)
    ent = p * jnp.log(jnp.maximum(p, 1e-10))
    s = jnp.sum(ent, axis=(0, 1), keepdims=True)
    o_ref[...] = jnp.exp(-s)


_perp = pl.pallas_call(
    _perp_body,
    out_shape=jax.ShapeDtypeStruct((1, 1), jnp.float32),
)


def kernel(z, codebook):
    B, C, H, W = z.shape
    flat_z = jnp.transpose(z, (0, 2, 3, 1)).reshape(-1, C)
    wt = codebook.T

    gather = _gather_kernel()
    idx0 = _argmin_first(flat_z, wt)
    q0 = gather(codebook, idx0.reshape(-1))
    idx1, r1 = _argmin_step(flat_z, q0, wt)
    q1 = gather(codebook, idx1.reshape(-1))
    idx2, r2 = _argmin_step(r1, q1, wt)
    q2 = gather(codebook, idx2.reshape(-1))
    idx3, r3 = _argmin_step(r2, q2, wt)
    q3 = gather(codebook, idx3.reshape(-1))

    r4 = r3 - q3
    cum1 = q1 + q0
    cum2 = q2 + cum1
    cum3 = q3 + cum2
    st = cum3 + (r4 - cum3)

    idx_all = jnp.concatenate(
        [idx0.reshape(-1), idx1.reshape(-1), idx2.reshape(-1), idx3.reshape(-1)],
        axis=0)
    counts = _bincount_kernel()(idx_all)
    perp = _perp(counts.reshape(64, 128)).reshape(())

    def unflat(x):
        return x.reshape(B, H, W, C).transpose(0, 3, 1, 2)

    return (unflat(q0), unflat(cum1), unflat(cum2), unflat(cum3), unflat(st),
            idx0.reshape(-1), idx1.reshape(-1), idx2.reshape(-1),
            idx3.reshape(-1), perp)
